# Initial kernel scaffold; baseline (speedup 1.0000x reference)
import jax
import jax.numpy as jnp
from jax.experimental import pallas as pl


def kernel(z_e, codebook):
    raise NotImplementedError("write your pallas kernel here")


# speedup vs baseline: 5.2516x; 5.2516x over previous
import jax
import jax.numpy as jnp
from jax.experimental import pallas as pl
from jax.experimental.pallas import tpu as pltpu

_N = 8192
_E = 8192
_D = 32

_RB = 512
_CB = 1024

_ERB = 512
_ECB = 1024


def _dist_argmin_kernel(z_ref, cb_ref, zsq_ref, esq_ref,
                        dist_ref, idx_ref, min_scr, arg_scr):
    j = pl.program_id(1)
    ncols = pl.num_programs(1)
    z = z_ref[...]
    cb = cb_ref[...]
    mm = jax.lax.dot_general(z, cb, (((1,), (1,)), ((), ())),
                             preferred_element_type=jnp.float32)
    d = (zsq_ref[...] + esq_ref[...]) - 2.0 * mm
    dist_ref[...] = d

    lmin = jnp.min(d, axis=1, keepdims=True)
    col = jax.lax.broadcasted_iota(jnp.int32, (_RB, _CB), 1) + j * _CB
    larg = jnp.min(jnp.where(d == lmin, col, jnp.int32(2**30)),
                   axis=1, keepdims=True)

    @pl.when(j == 0)
    def _init():
        min_scr[...] = lmin
        arg_scr[...] = larg

    @pl.when(j > 0)
    def _update():
        better = lmin < min_scr[...]
        arg_scr[...] = jnp.where(better, larg, arg_scr[...])
        min_scr[...] = jnp.where(better, lmin, min_scr[...])

    @pl.when(j == ncols - 1)
    def _emit():
        idx_ref[...] = arg_scr[...]


def _encodings_kernel(idx_ref, enc_ref):
    j = pl.program_id(1)
    col = jax.lax.broadcasted_iota(jnp.int32, (_ERB, _ECB), 1) + j * _ECB
    enc_ref[...] = (idx_ref[...] == col).astype(jnp.float32)


def kernel(z_e, codebook):
    z_e_nhwc = jnp.transpose(z_e, (0, 2, 3, 1))
    z_flat = z_e_nhwc.reshape(-1, _D)
    z_sq = jnp.sum(z_flat ** 2, axis=1, keepdims=True)
    e_sq = jnp.sum(codebook ** 2, axis=1).reshape(1, _E)

    dist, idx2d = pl.pallas_call(
        _dist_argmin_kernel,
        grid=(_N // _RB, _E // _CB),
        in_specs=[
            pl.BlockSpec((_RB, _D), lambda i, j: (i, 0)),
            pl.BlockSpec((_CB, _D), lambda i, j: (j, 0)),
            pl.BlockSpec((_RB, 1), lambda i, j: (i, 0)),
            pl.BlockSpec((1, _CB), lambda i, j: (0, j)),
        ],
        out_specs=[
            pl.BlockSpec((_RB, _CB), lambda i, j: (i, j)),
            pl.BlockSpec((_RB, 1), lambda i, j: (i, 0)),
        ],
        out_shape=[
            jax.ShapeDtypeStruct((_N, _E), jnp.float32),
            jax.ShapeDtypeStruct((_N, 1), jnp.int32),
        ],
        scratch_shapes=[
            pltpu.VMEM((_RB, 1), jnp.float32),
            pltpu.VMEM((_RB, 1), jnp.int32),
        ],
    )(z_flat, codebook, z_sq, e_sq)

    encodings = pl.pallas_call(
        _encodings_kernel,
        grid=(_N // _ERB, _E // _ECB),
        in_specs=[pl.BlockSpec((_ERB, 1), lambda i, j: (i, 0))],
        out_specs=pl.BlockSpec((_ERB, _ECB), lambda i, j: (i, j)),
        out_shape=jax.ShapeDtypeStruct((_N, _E), jnp.float32),
    )(idx2d)

    indices = idx2d.reshape(_N)
    return (z_e_nhwc, z_flat, dist, indices, encodings)
